# baseline (speedup 1.0000x reference)
import functools

import jax
import jax.numpy as jnp
import numpy as np
from jax import lax
from jax.experimental import pallas as pl
from jax.experimental.pallas import tpu as pltpu
from jax.experimental.pallas import tpu_sc as plsc

BATCH = 1024
MAX_LEN = 200
D_MODEL = 128
VOCAB = 1000
LANES = 16

NUM_TILES = 32
QBLOCKS = 8
TROWS = BATCH // QBLOCKS
NTASKS = MAX_LEN * QBLOCKS
TASKS_PER_TILE = NTASKS // NUM_TILES
POSROWS = 16
GSPLIT = 128


def _positional_encoding_np():
    even_i = np.arange(0, D_MODEL, 2, dtype=np.float64)
    denominator = np.power(10000.0, 2.0 * even_i / D_MODEL)
    position = np.arange(MAX_LEN, dtype=np.float64).reshape(MAX_LEN, 1)
    even_pe = np.sin(position / denominator)
    odd_pe = np.cos(position / denominator)
    stacked = np.stack([even_pe, odd_pe], axis=2)
    return stacked.reshape(MAX_LEN, D_MODEL).astype(np.float32)


_PE = _positional_encoding_np()


@functools.partial(
    pl.kernel,
    out_type=jax.ShapeDtypeStruct((BATCH, MAX_LEN, D_MODEL), jnp.float32),
    mesh=plsc.VectorSubcoreMesh(core_axis_name="c", subcore_axis_name="s"),
    compiler_params=pltpu.CompilerParams(use_tc_tiling_on_sc=False, skip_device_barrier=True),
    scratch_types=[
        pltpu.VMEM((POSROWS, BATCH), jnp.int32),
        pltpu.VMEM((POSROWS, D_MODEL), jnp.float32),
        pltpu.VMEM((TROWS, D_MODEL), jnp.float32),
        pltpu.VMEM((TROWS, D_MODEL), jnp.float32),
        pltpu.VMEM((TROWS, D_MODEL), jnp.float32),
        pltpu.VMEM((TROWS, D_MODEL), jnp.float32),
        pltpu.VMEM_SHARED((VOCAB, D_MODEL), jnp.float32),
        pltpu.SemaphoreType.DMA,
        pltpu.SemaphoreType.DMA,
        pltpu.SemaphoreType.DMA,
        pltpu.SemaphoreType.DMA,
        pltpu.SemaphoreType.DMA,
        pltpu.SemaphoreType.DMA,
        pltpu.SemaphoreType.DMA,
        pltpu.SemaphoreType.DMA,
    ],
)
def _emb_kernel(xt_hbm, table_hbm, pe_hbm, out_hbm, idx_v, pe_v, buf0, buf1,
                buf2, buf3, table_sp, g0, g1, g2, g3, s0, s1, s2, s3):
    sid = lax.axis_index("s")
    wid = sid * 2 + lax.axis_index("c")
    t0 = wid * TASKS_PER_TILE
    l0 = lax.shift_right_logical(t0, 3)
    l0c = jnp.minimum(
        lax.shift_left(lax.shift_right_logical(l0, 3), 3),
        MAX_LEN - POSROWS)

    @pl.when(sid == 0)
    def _():
        pltpu.sync_copy(table_hbm, table_sp)

    pltpu.sync_copy(xt_hbm.at[pl.ds(l0c, POSROWS)], idx_v)
    pltpu.sync_copy(pe_hbm.at[pl.ds(l0c, POSROWS)], pe_v)
    plsc.subcore_barrier()

    def task_lq(j):
        t = t0 + j
        l = lax.shift_right_logical(t, 3)
        q = lax.bitwise_and(t, 7)
        return l, q

    def gather_copies(j, buf, sem):
        l, q = task_lq(j)
        return [
            pltpu.make_async_copy(
                table_sp.at[idx_v.at[l - l0c,
                                     pl.ds(q * TROWS + h * GSPLIT, GSPLIT)]],
                buf.at[pl.ds(h * GSPLIT, GSPLIT)], sem)
            for h in range(TROWS // GSPLIT)
        ]

    def gather(j, buf, sem):
        for cp in gather_copies(j, buf, sem):
            cp.start()

    def gather_wait(j, buf, sem):
        for cp in gather_copies(j, buf, sem):
            cp.wait()

    def add_pe(j, buf):
        l, _ = task_lq(j)
        lr = l - l0c
        pe_regs = [pe_v[lr, pl.ds(k * LANES, LANES)]
                   for k in range(D_MODEL // LANES)]

        @plsc.parallel_loop(0, TROWS, step=1, unroll=4)
        def _(r):
            for k in range(D_MODEL // LANES):
                plsc.addupdate(buf.at[r, pl.ds(k * LANES, LANES)], pe_regs[k])

    def store(j, buf, sem):
        l, q = task_lq(j)
        pltpu.make_async_copy(
            buf, out_hbm.at[pl.ds(q * TROWS, TROWS), l], sem).start()

    def store_wait(j, buf, sem):
        l, q = task_lq(j)
        pltpu.make_async_copy(
            buf, out_hbm.at[pl.ds(q * TROWS, TROWS), l], sem).wait()

    bufs = (buf0, buf1, buf2, buf3)
    gsems = (g0, g1, g2, g3)
    ssems = (s0, s1, s2, s3)

    gather(0, buf0, g0)
    gather(1, buf1, g1)

    def step(j, kc, prefetch, guarded):
        bc, gc, sc = bufs[kc], gsems[kc], ssems[kc]
        kp = (kc + 2) % 4
        gather_wait(j, bc, gc)
        if guarded:
            @pl.when(j >= 2)
            def _():
                store_wait(j - 2, bufs[kp], ssems[kp])
        else:
            store_wait(j - 2, bufs[kp], ssems[kp])
        if prefetch:
            gather(j + 2, bufs[kp], gsems[kp])
        add_pe(j, bc)
        store(j, bc, sc)

    def quad_body(i, carry):
        j0 = 4 * i
        step(j0, 0, True, True)
        step(j0 + 1, 1, True, True)
        step(j0 + 2, 2, True, False)
        step(j0 + 3, 3, True, False)
        return carry

    lax.fori_loop(0, (TASKS_PER_TILE - 6) // 4, quad_body, 0)
    step(44, 0, True, False)
    step(45, 1, True, False)
    step(46, 2, True, False)
    step(47, 3, True, False)
    step(48, 0, False, False)
    step(49, 1, False, False)
    store_wait(48, buf0, s0)
    store_wait(49, buf1, s1)


def kernel(x, embedding_table):
    xt = jnp.transpose(x)
    pe = jnp.asarray(_PE)
    return _emb_kernel(xt, embedding_table, pe)


# speedup vs baseline: 1.0034x; 1.0034x over previous
import functools

import jax
import jax.numpy as jnp
import numpy as np
from jax import lax
from jax.experimental import pallas as pl
from jax.experimental.pallas import tpu as pltpu
from jax.experimental.pallas import tpu_sc as plsc

BATCH = 1024
MAX_LEN = 200
D_MODEL = 128
VOCAB = 1000
LANES = 16

NUM_TILES = 32
QBLOCKS = 8
TROWS = BATCH // QBLOCKS
NTASKS = MAX_LEN * QBLOCKS
TASKS_PER_TILE = NTASKS // NUM_TILES
POSROWS = 16
GSPLIT = 128


def _positional_encoding_np():
    even_i = np.arange(0, D_MODEL, 2, dtype=np.float64)
    denominator = np.power(10000.0, 2.0 * even_i / D_MODEL)
    position = np.arange(MAX_LEN, dtype=np.float64).reshape(MAX_LEN, 1)
    even_pe = np.sin(position / denominator)
    odd_pe = np.cos(position / denominator)
    stacked = np.stack([even_pe, odd_pe], axis=2)
    return stacked.reshape(MAX_LEN, D_MODEL).astype(np.float32)


_PE = _positional_encoding_np()


@functools.partial(
    pl.kernel,
    out_type=jax.ShapeDtypeStruct((BATCH, MAX_LEN, D_MODEL), jnp.float32),
    mesh=plsc.VectorSubcoreMesh(core_axis_name="c", subcore_axis_name="s"),
    compiler_params=pltpu.CompilerParams(use_tc_tiling_on_sc=False),
    scratch_types=[
        pltpu.VMEM((POSROWS, BATCH), jnp.int32),
        pltpu.VMEM((POSROWS, D_MODEL), jnp.float32),
        pltpu.VMEM((TROWS, D_MODEL), jnp.float32),
        pltpu.VMEM((TROWS, D_MODEL), jnp.float32),
        pltpu.VMEM((TROWS, D_MODEL), jnp.float32),
        pltpu.VMEM((TROWS, D_MODEL), jnp.float32),
        pltpu.VMEM_SHARED((VOCAB, D_MODEL), jnp.float32),
        pltpu.SemaphoreType.DMA,
        pltpu.SemaphoreType.DMA,
        pltpu.SemaphoreType.DMA,
        pltpu.SemaphoreType.DMA,
        pltpu.SemaphoreType.DMA,
        pltpu.SemaphoreType.DMA,
        pltpu.SemaphoreType.DMA,
        pltpu.SemaphoreType.DMA,
    ],
)
def _emb_kernel(xt_hbm, table_hbm, pe_hbm, out_hbm, idx_v, pe_v, buf0, buf1,
                buf2, buf3, table_sp, g0, g1, g2, g3, s0, s1, s2, s3):
    sid = lax.axis_index("s")
    wid = sid * 2 + lax.axis_index("c")
    t0 = wid * TASKS_PER_TILE
    l0 = lax.shift_right_logical(t0, 3)
    l0c = jnp.minimum(
        lax.shift_left(lax.shift_right_logical(l0, 3), 3),
        MAX_LEN - POSROWS)

    @pl.when(sid == 0)
    def _():
        pltpu.sync_copy(table_hbm, table_sp)

    pltpu.sync_copy(xt_hbm.at[pl.ds(l0c, POSROWS)], idx_v)
    pltpu.sync_copy(pe_hbm.at[pl.ds(l0c, POSROWS)], pe_v)
    plsc.subcore_barrier()

    def task_lq(j):
        t = t0 + j
        l = lax.shift_right_logical(t, 3)
        q = lax.bitwise_and(t, 7)
        return l, q

    def gather_copies(j, buf, sem):
        l, q = task_lq(j)
        return [
            pltpu.make_async_copy(
                table_sp.at[idx_v.at[l - l0c,
                                     pl.ds(q * TROWS + h * GSPLIT, GSPLIT)]],
                buf.at[pl.ds(h * GSPLIT, GSPLIT)], sem)
            for h in range(TROWS // GSPLIT)
        ]

    def gather(j, buf, sem):
        for cp in gather_copies(j, buf, sem):
            cp.start()

    def gather_wait(j, buf, sem):
        for cp in gather_copies(j, buf, sem):
            cp.wait()

    def add_pe(j, buf):
        l, _ = task_lq(j)
        lr = l - l0c
        pe_regs = [pe_v[lr, pl.ds(k * LANES, LANES)]
                   for k in range(D_MODEL // LANES)]

        @plsc.parallel_loop(0, TROWS, step=1, unroll=4)
        def _(r):
            for k in range(D_MODEL // LANES):
                plsc.addupdate(buf.at[r, pl.ds(k * LANES, LANES)], pe_regs[k])

    def store(j, buf, sem):
        l, q = task_lq(j)
        pltpu.make_async_copy(
            buf, out_hbm.at[pl.ds(q * TROWS, TROWS), l], sem).start()

    def store_wait(j, buf, sem):
        l, q = task_lq(j)
        pltpu.make_async_copy(
            buf, out_hbm.at[pl.ds(q * TROWS, TROWS), l], sem).wait()

    bufs = (buf0, buf1, buf2, buf3)
    gsems = (g0, g1, g2, g3)
    ssems = (s0, s1, s2, s3)

    gather(0, buf0, g0)
    gather(1, buf1, g1)

    def step(j, kc, prefetch, guarded):
        bc, gc, sc = bufs[kc], gsems[kc], ssems[kc]
        kp = (kc + 2) % 4
        gather_wait(j, bc, gc)
        if guarded:
            @pl.when(j >= 2)
            def _():
                store_wait(j - 2, bufs[kp], ssems[kp])
        else:
            store_wait(j - 2, bufs[kp], ssems[kp])
        if prefetch:
            gather(j + 2, bufs[kp], gsems[kp])
        add_pe(j, bc)
        store(j, bc, sc)

    def quad_body(i, carry):
        j0 = 4 * i
        step(j0, 0, True, True)
        step(j0 + 1, 1, True, True)
        step(j0 + 2, 2, True, False)
        step(j0 + 3, 3, True, False)
        return carry

    lax.fori_loop(0, (TASKS_PER_TILE - 6) // 4, quad_body, 0)
    step(44, 0, True, False)
    step(45, 1, True, False)
    step(46, 2, True, False)
    step(47, 3, True, False)
    step(48, 0, False, False)
    step(49, 1, False, False)
    store_wait(48, buf0, s0)
    store_wait(49, buf1, s1)


def kernel(x, embedding_table):
    xt = jnp.transpose(x)
    pe = jnp.asarray(_PE)
    return _emb_kernel(xt, embedding_table, pe)
